# baseline (speedup 1.0000x reference)
import functools
import math

import jax
import jax.numpy as jnp
import numpy as np
from jax import lax
from jax.experimental import pallas as pl
from jax.experimental.pallas import tpu as pltpu
from jax.experimental.pallas import tpu_sc as plsc

N = 10000
E = 320000
D = 128
INV_SQRT_DEG = 1.0 / math.sqrt(32.0)

NC = 2
NS = 16
NW = NC * NS

CP = 80
W_CH = E // (NW * CP)
QUADS = (W_CH - 1) // 4
ROWS_PER_SUB = 624
TAIL_ROWS = N - NS * ROWS_PER_SUB
STAGE_ROWS = 48
STAGE_ITERS = ROWS_PER_SUB // STAGE_ROWS

_DOT = dict(precision=lax.Precision.HIGHEST, preferred_element_type=jnp.float32)
BLK = 1000
BLKN = 2000

_PERM = np.empty(D, dtype=np.int32)
_PERM[0::2] = np.arange(64)
_PERM[1::2] = np.arange(64) + 64


def _node_body(x_ref, ws_ref, wd_ref, b_ref, a_ref, bm_ref):
    xb = x_ref[...]
    a_ref[...] = lax.dot_general(
        xb, ws_ref[...], (((1,), (0,)), ((), ())), **_DOT
    ).astype(jnp.bfloat16)
    bm_ref[...] = (
        lax.dot_general(xb, wd_ref[...], (((1,), (0,)), ((), ())), **_DOT)
        + b_ref[...]
    ).astype(jnp.bfloat16)


_node_call = pl.pallas_call(
    _node_body,
    grid=(N // BLKN,),
    in_specs=[
        pl.BlockSpec((BLKN, D), lambda i: (i, 0)),
        pl.BlockSpec((D, D), lambda i: (0, 0)),
        pl.BlockSpec((D, D), lambda i: (0, 0)),
        pl.BlockSpec((1, D), lambda i: (0, 0)),
    ],
    out_specs=[
        pl.BlockSpec((BLKN, D), lambda i: (i, 0)),
        pl.BlockSpec((BLKN, D), lambda i: (i, 0)),
    ],
    out_shape=[jax.ShapeDtypeStruct((N, D), jnp.bfloat16)] * 2,
)


_sc_mesh = plsc.VectorSubcoreMesh(
    core_axis_name="c", subcore_axis_name="s", num_cores=NC, num_subcores=NS
)


@functools.partial(
    pl.kernel,
    out_type=jax.ShapeDtypeStruct((NC, N, D), jnp.float32),
    mesh=_sc_mesh,
    compiler_params=pltpu.CompilerParams(use_tc_tiling_on_sc=False),
    scratch_types=[
        pltpu.VMEM((4, CP), jnp.int32),
        pltpu.VMEM((4, CP), jnp.int32),
        pltpu.VMEM((CP, D // 2), jnp.int32),
        pltpu.VMEM((CP, D // 2), jnp.int32),
        pltpu.VMEM((CP, D // 2), jnp.int32),
        pltpu.VMEM((CP, D // 2), jnp.int32),
        pltpu.VMEM((CP, D), jnp.float32),
        pltpu.VMEM((CP, D), jnp.float32),
        pltpu.VMEM((STAGE_ROWS, D), jnp.float32),
        pltpu.VMEM_SHARED((N, D), jnp.float32),
        pltpu.SemaphoreType.DMA,
        pltpu.SemaphoreType.DMA,
        pltpu.SemaphoreType.DMA,
        pltpu.SemaphoreType.DMA,
        pltpu.SemaphoreType.DMA,
    ],
)
def _edge_kernel(src_hbm, dst_hbm, a_hbm, b_hbm, zero_hbm, out_hbm,
                 idx_s4, idx_d4, a0, b0, a1, b1, o0, o1,
                 stage, acc, sem_i0, sem_i1, sem_g0, sem_g1, sem_s):
    c = lax.axis_index("c")
    s = lax.axis_index("s")
    wid = s * NC + c
    row0 = s * ROWS_PER_SUB

    idx_s = tuple(idx_s4.at[q] for q in range(4))
    idx_d = tuple(idx_d4.at[q] for q in range(4))
    abuf = (a0, a1)
    bbuf = (b0, b1)
    obuf = (o0, o1)
    sem_i = (sem_i0, sem_i1)
    sem_g = (sem_g0, sem_g1)

    def chunk_base(k):
        return (k * NW + wid) * CP

    def start_idx(k, q):
        base = chunk_base(k)
        pltpu.async_copy(src_hbm.at[pl.ds(base, CP)], idx_s[q], sem_i[q % 2])
        pltpu.async_copy(dst_hbm.at[pl.ds(base, CP)], idx_d[q], sem_i[q % 2])

    def wait_idx(k, q):
        base = chunk_base(k)
        pltpu.make_async_copy(src_hbm.at[pl.ds(base, CP)], idx_s[q], sem_i[q % 2]).wait()
        pltpu.make_async_copy(dst_hbm.at[pl.ds(base, CP)], idx_d[q], sem_i[q % 2]).wait()

    def start_gathers(p, q):
        pltpu.async_copy(a_hbm.at[idx_s[q]], abuf[p], sem_g[p])
        pltpu.async_copy(b_hbm.at[idx_d[q]], bbuf[p], sem_g[p])

    def wait_gathers(p, q):
        pltpu.make_async_copy(a_hbm.at[idx_s[q]], abuf[p], sem_g[p]).wait()
        pltpu.make_async_copy(b_hbm.at[idx_d[q]], bbuf[p], sem_g[p]).wait()

    def relu_part(p, lo, hi):
        ab, bb, ob = abuf[p], bbuf[p], obuf[p]
        himask = jnp.int32(-65536)

        @plsc.parallel_loop(lo, hi, step=1, unroll=4)
        def _(r):
            for g in range(4):
                sl = pl.ds(16 * g, 16)
                wa = ab[r, sl]
                wb = bb[r, sl]
                a_lo = lax.bitcast_convert_type(wa << 16, jnp.float32)
                b_lo = lax.bitcast_convert_type(wb << 16, jnp.float32)
                a_hi = lax.bitcast_convert_type(wa & himask, jnp.float32)
                b_hi = lax.bitcast_convert_type(wb & himask, jnp.float32)
                ob[r, pl.ds(16 * g, 16)] = jnp.maximum(a_lo + b_lo, 0.0)
                ob[r, pl.ds(64 + 16 * g, 16)] = jnp.maximum(a_hi + b_hi, 0.0)

    def scatter(p, q):
        pltpu.sync_copy(obuf[p], acc.at[idx_s[q]], add=True)

    def start_scatter(p, q):
        return pltpu.async_copy(obuf[p], acc.at[idx_s[q]], sem_s, add=True)

    def wait_scatter(p, q):
        pltpu.make_async_copy(obuf[p], acc.at[idx_s[q]], sem_s).wait()

    start_idx(0, 0)
    pltpu.sync_copy(zero_hbm.at[pl.ds(0, STAGE_ROWS)], stage)
    for t in range(STAGE_ITERS):
        pltpu.sync_copy(stage, acc.at[pl.ds(row0 + t * STAGE_ROWS, STAGE_ROWS)])

    @pl.when(s == NS - 1)
    def _():
        tail0 = NS * ROWS_PER_SUB
        pltpu.sync_copy(stage.at[pl.ds(0, TAIL_ROWS)], acc.at[pl.ds(tail0, TAIL_ROWS)])

    plsc.subcore_barrier()
    wait_idx(0, 0)
    start_gathers(0, 0)
    start_idx(1, 1)

    def step(k, p, q):
        wait_gathers(p, q)

        @pl.when(k > 0)
        def _():
            start_scatter(1 - p, (q + 3) % 4)

        relu_part(p, 0, CP // 2)

        @pl.when(k > 0)
        def _():
            wait_scatter(1 - p, (q + 3) % 4)

        @pl.when(k < W_CH - 2)
        def _():
            start_idx(k + 2, (q + 2) % 4)

        wait_idx(k + 1, (q + 1) % 4)
        start_gathers(1 - p, (q + 1) % 4)
        relu_part(p, CP // 2, CP)

    def quad_body(i, carry):
        k0 = i * 4
        step(k0, 0, 0)
        step(k0 + 1, 1, 1)
        step(k0 + 2, 0, 2)
        step(k0 + 3, 1, 3)
        return carry

    lax.fori_loop(0, QUADS, quad_body, 0)
    wait_gathers(0, 0)
    start_scatter(1, 3)
    relu_part(0, 0, CP // 2)
    wait_scatter(1, 3)
    relu_part(0, CP // 2, CP)
    scatter(0, 0)

    plsc.subcore_barrier()
    for k in range(STAGE_ITERS):
        r0 = row0 + k * STAGE_ROWS
        pltpu.sync_copy(acc.at[pl.ds(r0, STAGE_ROWS)], stage)
        pltpu.sync_copy(stage, out_hbm.at[c, pl.ds(r0, STAGE_ROWS)])

    @pl.when(s == NS - 1)
    def _():
        tail0 = NS * ROWS_PER_SUB
        pltpu.sync_copy(acc.at[pl.ds(tail0, TAIL_ROWS)], stage.at[pl.ds(0, TAIL_ROWS)])
        pltpu.sync_copy(stage.at[pl.ds(0, TAIL_ROWS)], out_hbm.at[c, pl.ds(tail0, TAIL_ROWS)])


def _update_body(x_ref, m_ref, wu_ref, b_ref, o_ref):
    xb = x_ref[...]
    msum = (m_ref[0] + m_ref[1]) * INV_SQRT_DEG
    h = (
        lax.dot_general(xb + msum, wu_ref[...], (((1,), (0,)), ((), ())), **_DOT)
        + b_ref[...]
    )
    o_ref[...] = xb + jnp.maximum(h, 0.0)


_update_call = pl.pallas_call(
    _update_body,
    grid=(N // BLK,),
    in_specs=[
        pl.BlockSpec((BLK, D), lambda i: (i, 0)),
        pl.BlockSpec((NC, BLK, D), lambda i: (0, i, 0)),
        pl.BlockSpec((D, D), lambda i: (0, 0)),
        pl.BlockSpec((1, D), lambda i: (0, 0)),
    ],
    out_specs=pl.BlockSpec((BLK, D), lambda i: (i, 0)),
    out_shape=jax.ShapeDtypeStruct((N, D), jnp.float32),
)


def kernel(x, edge_index, W_interact, b_interact, W_update, b_update):
    src = edge_index[0]
    dst = edge_index[1]
    a, bm = _node_call(
        x,
        W_interact[:D][:, _PERM],
        W_interact[D:][:, _PERM],
        b_interact[_PERM].reshape(1, D),
    )
    a32 = lax.bitcast_convert_type(a.reshape(N, D // 2, 2), jnp.int32)
    bm32 = lax.bitcast_convert_type(bm.reshape(N, D // 2, 2), jnp.int32)
    zeros = jnp.zeros((N, D), jnp.float32)
    mpair = _edge_kernel(src, dst, a32, bm32, zeros)
    return _update_call(x, mpair, W_update, b_update.reshape(1, D))


# speedup vs baseline: 1.0145x; 1.0145x over previous
import functools
import math

import jax
import jax.numpy as jnp
import numpy as np
from jax import lax
from jax.experimental import pallas as pl
from jax.experimental.pallas import tpu as pltpu
from jax.experimental.pallas import tpu_sc as plsc

N = 10000
E = 320000
D = 128
INV_SQRT_DEG = 1.0 / math.sqrt(32.0)

NC = 2
NS = 16
NW = NC * NS

CP = 80
W_CH = E // (NW * CP)
QUADS = (W_CH - 1) // 4
ROWS_PER_SUB = 624
TAIL_ROWS = N - NS * ROWS_PER_SUB
STAGE_ROWS = 48
STAGE_ITERS = ROWS_PER_SUB // STAGE_ROWS

_DOT = dict(precision=lax.Precision.HIGHEST, preferred_element_type=jnp.float32)
BLK = 1000
BLKN = 2000

_PERM = np.empty(D, dtype=np.int32)
_PERM[0::2] = np.arange(64)
_PERM[1::2] = np.arange(64) + 64


def _node_body(x_ref, ws_ref, wd_ref, b_ref, a_ref, bm_ref):
    xb = x_ref[...]
    a_ref[...] = lax.dot_general(
        xb, ws_ref[...], (((1,), (0,)), ((), ())), **_DOT
    ).astype(jnp.bfloat16)
    bm_ref[...] = (
        lax.dot_general(xb, wd_ref[...], (((1,), (0,)), ((), ())), **_DOT)
        + b_ref[...]
    ).astype(jnp.bfloat16)


_node_call = pl.pallas_call(
    _node_body,
    grid=(N // BLKN,),
    in_specs=[
        pl.BlockSpec((BLKN, D), lambda i: (i, 0)),
        pl.BlockSpec((D, D), lambda i: (0, 0)),
        pl.BlockSpec((D, D), lambda i: (0, 0)),
        pl.BlockSpec((1, D), lambda i: (0, 0)),
    ],
    out_specs=[
        pl.BlockSpec((BLKN, D), lambda i: (i, 0)),
        pl.BlockSpec((BLKN, D), lambda i: (i, 0)),
    ],
    out_shape=[jax.ShapeDtypeStruct((N, D), jnp.bfloat16)] * 2,
)


_sc_mesh = plsc.VectorSubcoreMesh(
    core_axis_name="c", subcore_axis_name="s", num_cores=NC, num_subcores=NS
)


@functools.partial(
    pl.kernel,
    out_type=jax.ShapeDtypeStruct((NC, N, D), jnp.float32),
    mesh=_sc_mesh,
    compiler_params=pltpu.CompilerParams(use_tc_tiling_on_sc=False),
    scratch_types=[
        pltpu.VMEM((4, CP), jnp.int32),
        pltpu.VMEM((4, CP), jnp.int32),
        pltpu.VMEM((CP, D // 2), jnp.int32),
        pltpu.VMEM((CP, D // 2), jnp.int32),
        pltpu.VMEM((CP, D // 2), jnp.int32),
        pltpu.VMEM((CP, D // 2), jnp.int32),
        pltpu.VMEM((CP, D), jnp.float32),
        pltpu.VMEM((CP, D), jnp.float32),
        pltpu.VMEM((STAGE_ROWS, D), jnp.float32),
        pltpu.VMEM_SHARED((N, D), jnp.float32),
        pltpu.SemaphoreType.DMA,
        pltpu.SemaphoreType.DMA,
        pltpu.SemaphoreType.DMA,
        pltpu.SemaphoreType.DMA,
    ],
)
def _edge_kernel(src_hbm, dst_hbm, a_hbm, b_hbm, zero_hbm, out_hbm,
                 idx_s4, idx_d4, a0, b0, a1, b1, o0, o1,
                 stage, acc, sem_i0, sem_i1, sem_g0, sem_g1):
    c = lax.axis_index("c")
    s = lax.axis_index("s")
    wid = s * NC + c
    row0 = s * ROWS_PER_SUB

    idx_s = tuple(idx_s4.at[q] for q in range(4))
    idx_d = tuple(idx_d4.at[q] for q in range(4))
    abuf = (a0, a1)
    bbuf = (b0, b1)
    obuf = (o0, o1)
    sem_i = (sem_i0, sem_i1)
    sem_g = (sem_g0, sem_g1)

    def chunk_base(k):
        return (k * NW + wid) * CP

    def start_idx(k, q):
        base = chunk_base(k)
        pltpu.async_copy(src_hbm.at[pl.ds(base, CP)], idx_s[q], sem_i[q % 2])
        pltpu.async_copy(dst_hbm.at[pl.ds(base, CP)], idx_d[q], sem_i[q % 2])

    def wait_idx(k, q):
        base = chunk_base(k)
        pltpu.make_async_copy(src_hbm.at[pl.ds(base, CP)], idx_s[q], sem_i[q % 2]).wait()
        pltpu.make_async_copy(dst_hbm.at[pl.ds(base, CP)], idx_d[q], sem_i[q % 2]).wait()

    def start_gathers(p, q):
        pltpu.async_copy(a_hbm.at[idx_s[q]], abuf[p], sem_g[p])
        pltpu.async_copy(b_hbm.at[idx_d[q]], bbuf[p], sem_g[p])

    def wait_gathers(p, q):
        pltpu.make_async_copy(a_hbm.at[idx_s[q]], abuf[p], sem_g[p]).wait()
        pltpu.make_async_copy(b_hbm.at[idx_d[q]], bbuf[p], sem_g[p]).wait()

    def relu_part(p, lo, hi):
        ab, bb, ob = abuf[p], bbuf[p], obuf[p]
        himask = jnp.int32(-65536)

        @plsc.parallel_loop(lo, hi, step=1, unroll=4)
        def _(r):
            for g in range(4):
                sl = pl.ds(16 * g, 16)
                wa = ab[r, sl]
                wb = bb[r, sl]
                a_lo = lax.bitcast_convert_type(wa << 16, jnp.float32)
                b_lo = lax.bitcast_convert_type(wb << 16, jnp.float32)
                a_hi = lax.bitcast_convert_type(wa & himask, jnp.float32)
                b_hi = lax.bitcast_convert_type(wb & himask, jnp.float32)
                ob[r, pl.ds(16 * g, 16)] = jnp.maximum(a_lo + b_lo, 0.0)
                ob[r, pl.ds(64 + 16 * g, 16)] = jnp.maximum(a_hi + b_hi, 0.0)

    def scatter(p, q):
        pltpu.sync_copy(obuf[p], acc.at[idx_s[q]], add=True)

    start_idx(0, 0)
    pltpu.sync_copy(zero_hbm.at[pl.ds(0, STAGE_ROWS)], stage)
    for t in range(STAGE_ITERS):
        pltpu.sync_copy(stage, acc.at[pl.ds(row0 + t * STAGE_ROWS, STAGE_ROWS)])

    @pl.when(s == NS - 1)
    def _():
        tail0 = NS * ROWS_PER_SUB
        pltpu.sync_copy(stage.at[pl.ds(0, TAIL_ROWS)], acc.at[pl.ds(tail0, TAIL_ROWS)])

    plsc.subcore_barrier()
    wait_idx(0, 0)
    start_gathers(0, 0)
    start_idx(1, 1)

    def step(k, p, q):
        wait_gathers(p, q)

        @pl.when(k > 0)
        def _():
            scatter(1 - p, (q + 3) % 4)

        @pl.when(k < W_CH - 2)
        def _():
            start_idx(k + 2, (q + 2) % 4)

        wait_idx(k + 1, (q + 1) % 4)
        start_gathers(1 - p, (q + 1) % 4)
        relu_part(p, 0, CP)

    def quad_body(i, carry):
        k0 = i * 4
        step(k0, 0, 0)
        step(k0 + 1, 1, 1)
        step(k0 + 2, 0, 2)
        step(k0 + 3, 1, 3)
        return carry

    lax.fori_loop(0, QUADS, quad_body, 0)
    wait_gathers(0, 0)
    scatter(1, 3)
    relu_part(0, 0, CP)
    scatter(0, 0)

    plsc.subcore_barrier()
    for k in range(STAGE_ITERS):
        r0 = row0 + k * STAGE_ROWS
        pltpu.sync_copy(acc.at[pl.ds(r0, STAGE_ROWS)], stage)
        pltpu.sync_copy(stage, out_hbm.at[c, pl.ds(r0, STAGE_ROWS)])

    @pl.when(s == NS - 1)
    def _():
        tail0 = NS * ROWS_PER_SUB
        pltpu.sync_copy(acc.at[pl.ds(tail0, TAIL_ROWS)], stage.at[pl.ds(0, TAIL_ROWS)])
        pltpu.sync_copy(stage.at[pl.ds(0, TAIL_ROWS)], out_hbm.at[c, pl.ds(tail0, TAIL_ROWS)])


def _update_body(x_ref, m_ref, wu_ref, b_ref, o_ref):
    xb = x_ref[...]
    msum = (m_ref[0] + m_ref[1]) * INV_SQRT_DEG
    h = (
        lax.dot_general(xb + msum, wu_ref[...], (((1,), (0,)), ((), ())), **_DOT)
        + b_ref[...]
    )
    o_ref[...] = xb + jnp.maximum(h, 0.0)


_update_call = pl.pallas_call(
    _update_body,
    grid=(N // BLK,),
    in_specs=[
        pl.BlockSpec((BLK, D), lambda i: (i, 0)),
        pl.BlockSpec((NC, BLK, D), lambda i: (0, i, 0)),
        pl.BlockSpec((D, D), lambda i: (0, 0)),
        pl.BlockSpec((1, D), lambda i: (0, 0)),
    ],
    out_specs=pl.BlockSpec((BLK, D), lambda i: (i, 0)),
    out_shape=jax.ShapeDtypeStruct((N, D), jnp.float32),
)


def kernel(x, edge_index, W_interact, b_interact, W_update, b_update):
    src = edge_index[0]
    dst = edge_index[1]
    a, bm = _node_call(
        x,
        W_interact[:D][:, _PERM],
        W_interact[D:][:, _PERM],
        b_interact[_PERM].reshape(1, D),
    )
    a32 = lax.bitcast_convert_type(a.reshape(N, D // 2, 2), jnp.int32)
    bm32 = lax.bitcast_convert_type(bm.reshape(N, D // 2, 2), jnp.int32)
    zeros = jnp.zeros((N, D), jnp.float32)
    mpair = _edge_kernel(src, dst, a32, bm32, zeros)
    return _update_call(x, mpair, W_update, b_update.reshape(1, D))
